# baseline (speedup 1.0000x reference)
import functools

import jax
import jax.numpy as jnp
from jax import lax
from jax.experimental import pallas as pl
from jax.experimental.pallas import tpu as pltpu
from jax.experimental.pallas import tpu_sc as plsc

N = 10000
E = 320000
D = 128
NC, NS = 2, 16
NW = NC * NS
EPW = E // NW
CH = 80
NCHUNK = EPW // CH
NPAD = 10240
RPT = NPAD // NS
RTC = 1024
GTC = (N + RTC - 1) // RTC


def _tc_log_mm(x_ref, w_ref, b_ref, o_ref):
    xb = x_ref[...]
    n = jnp.maximum(jnp.sqrt(jnp.sum(xb * xb, axis=1, keepdims=True)), 1e-7)
    u = (jnp.arctan2(n, 1.0) / n) * xb
    o_ref[...] = (
        jnp.dot(u, w_ref[...], preferred_element_type=jnp.float32) + b_ref[...]
    )


def _combine(h_ref, a0_ref, a1_ref, d_ref):
    a = a0_ref[...] + a1_ref[...]
    deg = d_ref[0] + d_ref[1]
    z = jnp.maximum(h_ref[...] + a / jnp.maximum(deg, 1.0), 0.0)
    s = jnp.sqrt(jnp.sum(z * z, axis=1, keepdims=True))
    return z / (s + 1e-7), s


def _tc_combine_mm(h_ref, a0_ref, a1_ref, d_ref, w_ref, b_ref, o_ref):
    z, _ = _combine(h_ref, a0_ref, a1_ref, d_ref)
    o_ref[...] = (
        jnp.dot(z, w_ref[...], preferred_element_type=jnp.float32) + b_ref[...]
    )


def _tc_combine_exp(h_ref, a0_ref, a1_ref, d_ref, o_ref):
    z, s = _combine(h_ref, a0_ref, a1_ref, d_ref)
    n = jnp.maximum(s / (s + 1e-7), 1e-7)
    o_ref[...] = (jnp.tan(n) / n) * z


def _transform(x, W, b):
    return pl.pallas_call(
        _tc_log_mm,
        grid=(GTC,),
        in_specs=[
            pl.BlockSpec((RTC, D), lambda i: (i, 0)),
            pl.BlockSpec((D, D), lambda i: (0, 0)),
            pl.BlockSpec((1, D), lambda i: (0, 0)),
        ],
        out_specs=pl.BlockSpec((RTC, D), lambda i: (i, 0)),
        out_shape=jax.ShapeDtypeStruct((N, D), jnp.float32),
    )(x, W, b.reshape(1, D))


_A0_SPEC = pl.BlockSpec((RTC, D), lambda i: (i, 0))
_A1_SPEC = pl.BlockSpec((RTC, D), lambda i: (NPAD // RTC + i, 0))
_DEG_SPEC = pl.BlockSpec((NC, RTC, 1), lambda i: (0, i, 0))


def _combine_mm(h, acc, deg, W, b):
    return pl.pallas_call(
        _tc_combine_mm,
        grid=(GTC,),
        in_specs=[
            pl.BlockSpec((RTC, D), lambda i: (i, 0)),
            _A0_SPEC,
            _A1_SPEC,
            _DEG_SPEC,
            pl.BlockSpec((D, D), lambda i: (0, 0)),
            pl.BlockSpec((1, D), lambda i: (0, 0)),
        ],
        out_specs=pl.BlockSpec((RTC, D), lambda i: (i, 0)),
        out_shape=jax.ShapeDtypeStruct((N, D), jnp.float32),
    )(h, acc, acc, deg.reshape(NC, NPAD, 1), W, b.reshape(1, D))


def _combine_exp(h, acc, deg):
    return pl.pallas_call(
        _tc_combine_exp,
        grid=(GTC,),
        in_specs=[
            pl.BlockSpec((RTC, D), lambda i: (i, 0)),
            _A0_SPEC,
            _A1_SPEC,
            _DEG_SPEC,
        ],
        out_specs=pl.BlockSpec((RTC, D), lambda i: (i, 0)),
        out_shape=jax.ShapeDtypeStruct((N, D), jnp.float32),
    )(h, acc, acc, deg.reshape(NC, NPAD, 1))


def _sc_segsum_kernel(h_hbm, src_hbm, dst_hbm, acc_out, deg_out,
                      src_idx, dst_idx, rows0, rows1, ones, zdeg,
                      acc, deg, sem0, sem1):
    cid = lax.axis_index("c")
    sid = lax.axis_index("s")
    wid = cid * NS + sid

    pltpu.sync_copy(src_hbm.at[pl.ds(wid * EPW, EPW)], src_idx)
    pltpu.sync_copy(dst_hbm.at[wid], dst_idx)

    zv = jnp.zeros((16,), jnp.float32)
    ov = jnp.ones((16,), jnp.float32)

    @pl.loop(0, CH // 16)
    def _fill_ones(i):
        ones[pl.ds(i * 16, 16)] = ov

    @pl.loop(0, CH)
    def _fill_zrow(r):
        for c in range(D // 16):
            rows0[r, pl.ds(c * 16, 16)] = zv

    @pl.loop(0, RPT // 16)
    def _fill_zdeg(i):
        zdeg[pl.ds(i * 16, 16)] = zv

    base = sid * RPT
    for i in range(RPT // CH):
        pltpu.sync_copy(rows0, acc.at[pl.ds(base + i * CH, CH)])
    pltpu.sync_copy(zdeg, deg.at[pl.ds(base, RPT)])
    plsc.subcore_barrier()

    def g_src(c):
        return h_hbm.at[src_idx.at[pl.ds(c * CH, CH)]]

    pltpu.async_copy(g_src(0), rows0, sem0)

    @pl.loop(0, NCHUNK - 1, step=2)
    def _chunks(c):
        d1 = pltpu.async_copy(g_src(c + 1), rows1, sem1)
        pltpu.make_async_copy(g_src(c), rows0, sem0).wait()
        pltpu.sync_copy(rows0, acc.at[dst_idx.at[c]], add=True)
        pltpu.async_copy(g_src(c + 2), rows0, sem0)
        pltpu.sync_copy(ones, deg.at[dst_idx.at[c]], add=True)
        d1.wait()
        pltpu.sync_copy(rows1, acc.at[dst_idx.at[c + 1]], add=True)
        pltpu.sync_copy(ones, deg.at[dst_idx.at[c + 1]], add=True)

    pltpu.make_async_copy(g_src(NCHUNK - 1), rows0, sem0).wait()
    pltpu.sync_copy(rows0, acc.at[dst_idx.at[NCHUNK - 1]], add=True)
    pltpu.sync_copy(ones, deg.at[dst_idx.at[NCHUNK - 1]], add=True)

    plsc.subcore_barrier()
    pltpu.sync_copy(acc.at[pl.ds(base, RPT)],
                    acc_out.at[pl.ds(cid * NPAD + base, RPT)])
    pltpu.sync_copy(deg.at[pl.ds(base, RPT)], deg_out.at[cid, pl.ds(base, RPT)])


_sc_segsum = pl.kernel(
    _sc_segsum_kernel,
    out_type=(
        jax.ShapeDtypeStruct((NC * NPAD, D), jnp.float32),
        jax.ShapeDtypeStruct((NC, NPAD), jnp.float32),
    ),
    mesh=plsc.VectorSubcoreMesh(
        core_axis_name="c", subcore_axis_name="s", num_cores=NC, num_subcores=NS
    ),
    scratch_types=(
        pltpu.VMEM((EPW,), jnp.int32),
        pltpu.VMEM((NCHUNK, CH), jnp.int32),
        pltpu.VMEM((CH, D), jnp.float32),
        pltpu.VMEM((CH, D), jnp.float32),
        pltpu.VMEM((CH,), jnp.float32),
        pltpu.VMEM((RPT,), jnp.float32),
        pltpu.VMEM_SHARED((NPAD, D), jnp.float32),
        pltpu.VMEM_SHARED((NPAD,), jnp.float32),
        pltpu.SemaphoreType.DMA,
        pltpu.SemaphoreType.DMA,
    ),
)


def kernel(x, adj, W1, b1, W2, b2):
    src1 = adj[0, 0]
    dst1 = adj[0, 1].reshape(NW, NCHUNK, CH)
    src2 = adj[1, 0]
    dst2 = adj[1, 1].reshape(NW, NCHUNK, CH)

    h1 = _transform(x, W1, b1)
    acc1, deg1 = _sc_segsum(h1, src1, dst1)
    h2 = _combine_mm(h1, acc1, deg1, W2, b2)
    acc2, deg2 = _sc_segsum(h2, src2, dst2)
    return _combine_exp(h2, acc2, deg2)


# speedup vs baseline: 1.0986x; 1.0986x over previous
import functools

import jax
import jax.numpy as jnp
from jax import lax
from jax.experimental import pallas as pl
from jax.experimental.pallas import tpu as pltpu
from jax.experimental.pallas import tpu_sc as plsc

N = 10000
E = 320000
D = 128
NC, NS = 2, 16
NW = NC * NS
EPW = E // NW
CH = 80
NCHUNK = EPW // CH
NPAD = 10240
RPT = NPAD // NS
RTC = 1024
GTC = (N + RTC - 1) // RTC


def _tc_log_mm(x_ref, w_ref, b_ref, o_ref):
    xb = x_ref[...]
    n = jnp.maximum(jnp.sqrt(jnp.sum(xb * xb, axis=1, keepdims=True)), 1e-7)
    u = (jnp.arctan2(n, 1.0) / n) * xb
    o_ref[...] = (
        jnp.dot(u, w_ref[...], preferred_element_type=jnp.float32) + b_ref[...]
    )


def _combine(h_ref, a0_ref, a1_ref, d_ref):
    a = a0_ref[...] + a1_ref[...]
    deg = (d_ref[0] + d_ref[1]).reshape(RTC, 1)
    z = jnp.maximum(h_ref[...] + a / jnp.maximum(deg, 1.0), 0.0)
    s = jnp.sqrt(jnp.sum(z * z, axis=1, keepdims=True))
    return z / (s + 1e-7), s


def _tc_combine_mm(h_ref, a0_ref, a1_ref, d_ref, w_ref, b_ref, o_ref):
    z, _ = _combine(h_ref, a0_ref, a1_ref, d_ref)
    o_ref[...] = (
        jnp.dot(z, w_ref[...], preferred_element_type=jnp.float32) + b_ref[...]
    )


def _tc_combine_exp(h_ref, a0_ref, a1_ref, d_ref, o_ref):
    z, s = _combine(h_ref, a0_ref, a1_ref, d_ref)
    n = s / (s + 1e-7)
    n2 = n * n
    n4 = n2 * n2
    factor = (945.0 - 105.0 * n2 + n4) / (945.0 - 420.0 * n2 + 15.0 * n4)
    o_ref[...] = factor * z


def _transform(x, W, b):
    return pl.pallas_call(
        _tc_log_mm,
        grid=(GTC,),
        in_specs=[
            pl.BlockSpec((RTC, D), lambda i: (i, 0)),
            pl.BlockSpec((D, D), lambda i: (0, 0)),
            pl.BlockSpec((1, D), lambda i: (0, 0)),
        ],
        out_specs=pl.BlockSpec((RTC, D), lambda i: (i, 0)),
        out_shape=jax.ShapeDtypeStruct((N, D), jnp.float32),
    )(x, W, b.reshape(1, D))


_A0_SPEC = pl.BlockSpec((RTC, D), lambda i: (i, 0))
_A1_SPEC = pl.BlockSpec((RTC, D), lambda i: (NPAD // RTC + i, 0))
_DEG_SPEC = pl.BlockSpec((NC, RTC), lambda i: (0, i))


def _combine_mm(h, acc, deg, W, b):
    return pl.pallas_call(
        _tc_combine_mm,
        grid=(GTC,),
        in_specs=[
            pl.BlockSpec((RTC, D), lambda i: (i, 0)),
            _A0_SPEC,
            _A1_SPEC,
            _DEG_SPEC,
            pl.BlockSpec((D, D), lambda i: (0, 0)),
            pl.BlockSpec((1, D), lambda i: (0, 0)),
        ],
        out_specs=pl.BlockSpec((RTC, D), lambda i: (i, 0)),
        out_shape=jax.ShapeDtypeStruct((N, D), jnp.float32),
    )(h, acc, acc, deg, W, b.reshape(1, D))


def _combine_exp(h, acc, deg):
    return pl.pallas_call(
        _tc_combine_exp,
        grid=(GTC,),
        in_specs=[
            pl.BlockSpec((RTC, D), lambda i: (i, 0)),
            _A0_SPEC,
            _A1_SPEC,
            _DEG_SPEC,
        ],
        out_specs=pl.BlockSpec((RTC, D), lambda i: (i, 0)),
        out_shape=jax.ShapeDtypeStruct((N, D), jnp.float32),
    )(h, acc, acc, deg)


def _sc_segsum_kernel(h_hbm, src_hbm, dst_hbm, acc_out, deg_out,
                      src_idx, dst_idx, rows0, rows1, ones, zdeg,
                      acc, deg, sem0, sem1, sem2):
    cid = lax.axis_index("c")
    sid = lax.axis_index("s")
    wid = cid * NS + sid

    pltpu.sync_copy(src_hbm.at[pl.ds(wid * EPW, EPW)], src_idx)

    zv = jnp.zeros((16,), jnp.float32)
    ov = jnp.ones((16,), jnp.float32)

    @pl.loop(0, CH // 16)
    def _fill_ones(i):
        ones[pl.ds(i * 16, 16)] = ov

    @pl.loop(0, CH)
    def _fill_zrow(r):
        for c in range(D // 16):
            rows0[r, pl.ds(c * 16, 16)] = zv

    @pl.loop(0, RPT // 16)
    def _fill_zdeg(i):
        zdeg[pl.ds(i * 16, 16)] = zv

    base = sid * RPT
    for i in range(RPT // CH):
        pltpu.sync_copy(rows0, acc.at[pl.ds(base + i * CH, CH)])
    pltpu.sync_copy(zdeg, deg.at[pl.ds(base, RPT)])
    plsc.subcore_barrier()

    def g_src(c):
        return h_hbm.at[src_idx.at[pl.ds(c * CH, CH)]]

    def d_src(c):
        return dst_hbm.at[pl.ds(wid * EPW + c * CH, CH)]

    pltpu.sync_copy(d_src(0), dst_idx.at[0])
    pltpu.sync_copy(d_src(1), dst_idx.at[1])
    pltpu.async_copy(g_src(0), rows0, sem0)

    @pl.loop(0, NCHUNK - 1, step=2)
    def _chunks(c):
        d1 = pltpu.async_copy(g_src(c + 1), rows1, sem1)
        pltpu.async_copy(d_src(c + 2), dst_idx.at[c + 2], sem2)
        pltpu.make_async_copy(g_src(c), rows0, sem0).wait()
        pltpu.sync_copy(rows0, acc.at[dst_idx.at[c]], add=True)
        pltpu.async_copy(g_src(c + 2), rows0, sem0)
        pltpu.sync_copy(ones, deg.at[dst_idx.at[c]], add=True)

        @pl.when(c + 3 < NCHUNK)
        def _():
            pltpu.async_copy(d_src(c + 3), dst_idx.at[c + 3], sem2)

        d1.wait()
        pltpu.sync_copy(rows1, acc.at[dst_idx.at[c + 1]], add=True)
        pltpu.sync_copy(ones, deg.at[dst_idx.at[c + 1]], add=True)
        pltpu.make_async_copy(d_src(c + 2), dst_idx.at[c + 2], sem2).wait()

        @pl.when(c + 3 < NCHUNK)
        def _():
            pltpu.make_async_copy(d_src(c + 3), dst_idx.at[c + 3], sem2).wait()

    pltpu.make_async_copy(g_src(NCHUNK - 1), rows0, sem0).wait()
    pltpu.sync_copy(rows0, acc.at[dst_idx.at[NCHUNK - 1]], add=True)
    pltpu.sync_copy(ones, deg.at[dst_idx.at[NCHUNK - 1]], add=True)

    plsc.subcore_barrier()
    pltpu.sync_copy(acc.at[pl.ds(base, RPT)],
                    acc_out.at[pl.ds(cid * NPAD + base, RPT)])
    pltpu.sync_copy(deg.at[pl.ds(base, RPT)], deg_out.at[cid, pl.ds(base, RPT)])


_sc_segsum = pl.kernel(
    _sc_segsum_kernel,
    out_type=(
        jax.ShapeDtypeStruct((NC * NPAD, D), jnp.float32),
        jax.ShapeDtypeStruct((NC, NPAD), jnp.float32),
    ),
    mesh=plsc.VectorSubcoreMesh(
        core_axis_name="c", subcore_axis_name="s", num_cores=NC, num_subcores=NS
    ),
    scratch_types=(
        pltpu.VMEM((EPW,), jnp.int32),
        pltpu.VMEM((NCHUNK, CH), jnp.int32),
        pltpu.VMEM((CH, D), jnp.float32),
        pltpu.VMEM((CH, D), jnp.float32),
        pltpu.VMEM((CH,), jnp.float32),
        pltpu.VMEM((RPT,), jnp.float32),
        pltpu.VMEM_SHARED((NPAD, D), jnp.float32),
        pltpu.VMEM_SHARED((NPAD,), jnp.float32),
        pltpu.SemaphoreType.DMA,
        pltpu.SemaphoreType.DMA,
        pltpu.SemaphoreType.DMA,
    ),
)


def kernel(x, adj, W1, b1, W2, b2):
    src1 = adj[0, 0]
    dst1 = adj[0, 1]
    src2 = adj[1, 0]
    dst2 = adj[1, 1]

    h1 = _transform(x, W1, b1)
    acc1, deg1 = _sc_segsum(h1, src1, dst1)
    h2 = _combine_mm(h1, acc1, deg1, W2, b2)
    acc2, deg2 = _sc_segsum(h2, src2, dst2)
    return _combine_exp(h2, acc2, deg2)


# speedup vs baseline: 1.1259x; 1.0248x over previous
import functools

import jax
import jax.numpy as jnp
from jax import lax
from jax.experimental import pallas as pl
from jax.experimental.pallas import tpu as pltpu
from jax.experimental.pallas import tpu_sc as plsc

N = 10000
E = 320000
D = 128
NC, NS = 2, 16
NW = NC * NS
EPW = E // NW
CH = 80
NCHUNK = EPW // CH
NPAD = 10240
RPT = NPAD // NS
RTC = 1024
GTC = (N + RTC - 1) // RTC


def _tc_log_mm(x_ref, w_ref, b_ref, o_ref):
    xb = x_ref[...]
    n = jnp.maximum(jnp.sqrt(jnp.sum(xb * xb, axis=1, keepdims=True)), 1e-7)
    u = (jnp.arctan2(n, 1.0) / n) * xb
    o_ref[...] = (
        jnp.dot(u, w_ref[...], preferred_element_type=jnp.float32) + b_ref[...]
    )


def _combine(h_ref, a0_ref, a1_ref, d_ref):
    a = a0_ref[...] + a1_ref[...]
    deg = (d_ref[0] + d_ref[1]).reshape(RTC, 1)
    z = jnp.maximum(h_ref[...] + a / jnp.maximum(deg, 1.0), 0.0)
    s = jnp.sqrt(jnp.sum(z * z, axis=1, keepdims=True))
    return z / (s + 1e-7), s


def _tc_combine_mm(h_ref, a0_ref, a1_ref, d_ref, w_ref, b_ref, o_ref):
    z, _ = _combine(h_ref, a0_ref, a1_ref, d_ref)
    o_ref[...] = (
        jnp.dot(z, w_ref[...], preferred_element_type=jnp.float32) + b_ref[...]
    )


def _tc_combine_exp(h_ref, a0_ref, a1_ref, d_ref, o_ref):
    z, s = _combine(h_ref, a0_ref, a1_ref, d_ref)
    n = s / (s + 1e-7)
    n2 = n * n
    n4 = n2 * n2
    factor = (945.0 - 105.0 * n2 + n4) / (945.0 - 420.0 * n2 + 15.0 * n4)
    o_ref[...] = factor * z


def _transform(x, W, b):
    return pl.pallas_call(
        _tc_log_mm,
        grid=(GTC,),
        in_specs=[
            pl.BlockSpec((RTC, D), lambda i: (i, 0)),
            pl.BlockSpec((D, D), lambda i: (0, 0)),
            pl.BlockSpec((1, D), lambda i: (0, 0)),
        ],
        out_specs=pl.BlockSpec((RTC, D), lambda i: (i, 0)),
        out_shape=jax.ShapeDtypeStruct((N, D), jnp.float32),
    )(x, W, b.reshape(1, D))


_A0_SPEC = pl.BlockSpec((RTC, D), lambda i: (i, 0))
_A1_SPEC = pl.BlockSpec((RTC, D), lambda i: (NPAD // RTC + i, 0))
_DEG_SPEC = pl.BlockSpec((NC, RTC), lambda i: (0, i))


def _combine_mm(h, acc, deg, W, b):
    return pl.pallas_call(
        _tc_combine_mm,
        grid=(GTC,),
        in_specs=[
            pl.BlockSpec((RTC, D), lambda i: (i, 0)),
            _A0_SPEC,
            _A1_SPEC,
            _DEG_SPEC,
            pl.BlockSpec((D, D), lambda i: (0, 0)),
            pl.BlockSpec((1, D), lambda i: (0, 0)),
        ],
        out_specs=pl.BlockSpec((RTC, D), lambda i: (i, 0)),
        out_shape=jax.ShapeDtypeStruct((N, D), jnp.float32),
    )(h, acc, acc, deg, W, b.reshape(1, D))


def _combine_exp(h, acc, deg):
    return pl.pallas_call(
        _tc_combine_exp,
        grid=(GTC,),
        in_specs=[
            pl.BlockSpec((RTC, D), lambda i: (i, 0)),
            _A0_SPEC,
            _A1_SPEC,
            _DEG_SPEC,
        ],
        out_specs=pl.BlockSpec((RTC, D), lambda i: (i, 0)),
        out_shape=jax.ShapeDtypeStruct((N, D), jnp.float32),
    )(h, acc, acc, deg)


def _sc_segsum_kernel(layer, h_hbm, adj_hbm, acc_out, deg_out,
                      src_idx, dst_idx, rows0, rows1, ones, zdeg,
                      acc, deg, sem0, sem1, sem2):
    cid = lax.axis_index("c")
    sid = lax.axis_index("s")
    wid = cid * NS + sid
    src_base = (2 * layer) * E + wid * EPW
    dst_base = (2 * layer + 1) * E + wid * EPW

    pltpu.sync_copy(adj_hbm.at[pl.ds(src_base, EPW)], src_idx)

    zv = jnp.zeros((16,), jnp.float32)
    ov = jnp.ones((16,), jnp.float32)

    @pl.loop(0, CH // 16)
    def _fill_ones(i):
        ones[pl.ds(i * 16, 16)] = ov

    @pl.loop(0, CH)
    def _fill_zrow(r):
        for c in range(D // 16):
            rows0[r, pl.ds(c * 16, 16)] = zv

    @pl.loop(0, RPT // 16)
    def _fill_zdeg(i):
        zdeg[pl.ds(i * 16, 16)] = zv

    base = sid * RPT
    for i in range(RPT // CH):
        pltpu.sync_copy(rows0, acc.at[pl.ds(base + i * CH, CH)])
    pltpu.sync_copy(zdeg, deg.at[pl.ds(base, RPT)])
    plsc.subcore_barrier()

    def g_src(c):
        return h_hbm.at[src_idx.at[pl.ds(c * CH, CH)]]

    def d_src(c):
        return adj_hbm.at[pl.ds(dst_base + c * CH, CH)]

    pltpu.sync_copy(d_src(0), dst_idx.at[0])
    pltpu.sync_copy(d_src(1), dst_idx.at[1])
    pltpu.async_copy(g_src(0), rows0, sem0)

    @pl.loop(0, NCHUNK - 1, step=2)
    def _chunks(c):
        d1 = pltpu.async_copy(g_src(c + 1), rows1, sem1)
        pltpu.async_copy(d_src(c + 2), dst_idx.at[c + 2], sem2)
        pltpu.make_async_copy(g_src(c), rows0, sem0).wait()
        pltpu.sync_copy(rows0, acc.at[dst_idx.at[c]], add=True)
        pltpu.async_copy(g_src(c + 2), rows0, sem0)
        pltpu.sync_copy(ones, deg.at[dst_idx.at[c]], add=True)

        @pl.when(c + 3 < NCHUNK)
        def _():
            pltpu.async_copy(d_src(c + 3), dst_idx.at[c + 3], sem2)

        d1.wait()
        pltpu.sync_copy(rows1, acc.at[dst_idx.at[c + 1]], add=True)
        pltpu.sync_copy(ones, deg.at[dst_idx.at[c + 1]], add=True)
        pltpu.make_async_copy(d_src(c + 2), dst_idx.at[c + 2], sem2).wait()

        @pl.when(c + 3 < NCHUNK)
        def _():
            pltpu.make_async_copy(d_src(c + 3), dst_idx.at[c + 3], sem2).wait()

    pltpu.make_async_copy(g_src(NCHUNK - 1), rows0, sem0).wait()
    pltpu.sync_copy(rows0, acc.at[dst_idx.at[NCHUNK - 1]], add=True)
    pltpu.sync_copy(ones, deg.at[dst_idx.at[NCHUNK - 1]], add=True)

    plsc.subcore_barrier()
    pltpu.sync_copy(acc.at[pl.ds(base, RPT)],
                    acc_out.at[pl.ds(cid * NPAD + base, RPT)])
    pltpu.sync_copy(deg.at[pl.ds(base, RPT)], deg_out.at[cid, pl.ds(base, RPT)])


def _make_sc_segsum(layer):
    return pl.kernel(
        functools.partial(_sc_segsum_kernel, layer),
        out_type=(
            jax.ShapeDtypeStruct((NC * NPAD, D), jnp.float32),
            jax.ShapeDtypeStruct((NC, NPAD), jnp.float32),
        ),
        mesh=plsc.VectorSubcoreMesh(
            core_axis_name="c", subcore_axis_name="s",
            num_cores=NC, num_subcores=NS,
        ),
        scratch_types=(
            pltpu.VMEM((EPW,), jnp.int32),
            pltpu.VMEM((NCHUNK, CH), jnp.int32),
            pltpu.VMEM((CH, D), jnp.float32),
            pltpu.VMEM((CH, D), jnp.float32),
            pltpu.VMEM((CH,), jnp.float32),
            pltpu.VMEM((RPT,), jnp.float32),
            pltpu.VMEM_SHARED((NPAD, D), jnp.float32),
            pltpu.VMEM_SHARED((NPAD,), jnp.float32),
            pltpu.SemaphoreType.DMA,
            pltpu.SemaphoreType.DMA,
            pltpu.SemaphoreType.DMA,
        ),
    )


_sc_segsum_l0 = _make_sc_segsum(0)
_sc_segsum_l1 = _make_sc_segsum(1)


def kernel(x, adj, W1, b1, W2, b2):
    adj_flat = adj.reshape(2 * 2 * E)

    h1 = _transform(x, W1, b1)
    acc1, deg1 = _sc_segsum_l0(h1, adj_flat)
    h2 = _combine_mm(h1, acc1, deg1, W2, b2)
    acc2, deg2 = _sc_segsum_l1(h2, adj_flat)
    return _combine_exp(h2, acc2, deg2)


# speedup vs baseline: 1.1598x; 1.0301x over previous
import functools

import jax
import jax.numpy as jnp
from jax import lax
from jax.experimental import pallas as pl
from jax.experimental.pallas import tpu as pltpu
from jax.experimental.pallas import tpu_sc as plsc

N = 10000
E = 320000
D = 128
NC, NS = 2, 16
NW = NC * NS
EPW = E // NW
CH = 80
NCHUNK = EPW // CH
NPAD = 10240
RPT = NPAD // NS
RTC = 1024
GTC = (N + RTC - 1) // RTC


def _tc_log_mm(x_ref, w_ref, b_ref, o_ref):
    xb = x_ref[...]
    n = jnp.maximum(jnp.sqrt(jnp.sum(xb * xb, axis=1, keepdims=True)), 1e-7)
    u = (jnp.arctan2(n, 1.0) / n) * xb
    o_ref[...] = (
        jnp.dot(u, w_ref[...], preferred_element_type=jnp.float32) + b_ref[...]
    )


def _combine(h_ref, a0_ref, a1_ref, d_ref):
    a = a0_ref[...] + a1_ref[...]
    deg = (d_ref[0] + d_ref[1]).reshape(RTC, 1)
    z = jnp.maximum(h_ref[...] + a / jnp.maximum(deg, 1.0), 0.0)
    s = jnp.sqrt(jnp.sum(z * z, axis=1, keepdims=True))
    return z / (s + 1e-7), s


def _tc_combine_mm(h_ref, a0_ref, a1_ref, d_ref, w_ref, b_ref, o_ref):
    z, _ = _combine(h_ref, a0_ref, a1_ref, d_ref)
    o_ref[...] = (
        jnp.dot(z, w_ref[...], preferred_element_type=jnp.float32) + b_ref[...]
    )


def _tc_combine_exp(h_ref, a0_ref, a1_ref, d_ref, o_ref):
    z, s = _combine(h_ref, a0_ref, a1_ref, d_ref)
    n = s / (s + 1e-7)
    n2 = n * n
    n4 = n2 * n2
    factor = (945.0 - 105.0 * n2 + n4) / (945.0 - 420.0 * n2 + 15.0 * n4)
    o_ref[...] = factor * z


def _transform(x, W, b):
    return pl.pallas_call(
        _tc_log_mm,
        grid=(GTC,),
        in_specs=[
            pl.BlockSpec((RTC, D), lambda i: (i, 0)),
            pl.BlockSpec((D, D), lambda i: (0, 0)),
            pl.BlockSpec((1, D), lambda i: (0, 0)),
        ],
        out_specs=pl.BlockSpec((RTC, D), lambda i: (i, 0)),
        out_shape=jax.ShapeDtypeStruct((N, D), jnp.float32),
    )(x, W, b.reshape(1, D))


_A0_SPEC = pl.BlockSpec((RTC, D), lambda i: (i, 0))
_A1_SPEC = pl.BlockSpec((RTC, D), lambda i: (NPAD // RTC + i, 0))
_DEG_SPEC = pl.BlockSpec((NC, RTC), lambda i: (0, i))


def _combine_mm(h, acc, deg, W, b):
    return pl.pallas_call(
        _tc_combine_mm,
        grid=(GTC,),
        in_specs=[
            pl.BlockSpec((RTC, D), lambda i: (i, 0)),
            _A0_SPEC,
            _A1_SPEC,
            _DEG_SPEC,
            pl.BlockSpec((D, D), lambda i: (0, 0)),
            pl.BlockSpec((1, D), lambda i: (0, 0)),
        ],
        out_specs=pl.BlockSpec((RTC, D), lambda i: (i, 0)),
        out_shape=jax.ShapeDtypeStruct((N, D), jnp.float32),
    )(h, acc, acc, deg, W, b.reshape(1, D))


def _combine_exp(h, acc, deg):
    return pl.pallas_call(
        _tc_combine_exp,
        grid=(GTC,),
        in_specs=[
            pl.BlockSpec((RTC, D), lambda i: (i, 0)),
            _A0_SPEC,
            _A1_SPEC,
            _DEG_SPEC,
        ],
        out_specs=pl.BlockSpec((RTC, D), lambda i: (i, 0)),
        out_shape=jax.ShapeDtypeStruct((N, D), jnp.float32),
    )(h, acc, acc, deg)


def _sc_segsum_kernel(layer, h_hbm, adj_hbm, acc_out, deg_out,
                      src_idx, dst_idx, rows0, rows1, ones, zdeg,
                      acc, deg, sem0, sem1, sem2, sem3):
    cid = lax.axis_index("c")
    sid = lax.axis_index("s")
    wid = cid * NS + sid
    src_base = (2 * layer) * E + wid * EPW
    dst_base = (2 * layer + 1) * E + wid * EPW

    pltpu.sync_copy(adj_hbm.at[pl.ds(src_base, EPW)], src_idx)

    zv = jnp.zeros((16,), jnp.float32)
    ov = jnp.ones((16,), jnp.float32)

    @pl.loop(0, CH // 16)
    def _fill_ones(i):
        ones[pl.ds(i * 16, 16)] = ov

    @pl.loop(0, CH)
    def _fill_zrow(r):
        for c in range(D // 16):
            rows0[r, pl.ds(c * 16, 16)] = zv

    @pl.loop(0, RPT // 16)
    def _fill_zdeg(i):
        zdeg[pl.ds(i * 16, 16)] = zv

    base = sid * RPT
    for i in range(RPT // CH):
        pltpu.sync_copy(rows0, acc.at[pl.ds(base + i * CH, CH)])
    pltpu.sync_copy(zdeg, deg.at[pl.ds(base, RPT)])
    plsc.subcore_barrier()

    def g_src(c):
        return h_hbm.at[src_idx.at[pl.ds(c * CH, CH)]]

    def d_src(c):
        return adj_hbm.at[pl.ds(dst_base + c * CH, CH)]

    pltpu.sync_copy(d_src(0), dst_idx.at[0])
    pltpu.sync_copy(d_src(1), dst_idx.at[1])
    pltpu.async_copy(g_src(0), rows0, sem0)

    def w_deg(c):
        pltpu.make_async_copy(ones, deg.at[dst_idx.at[c]], sem3).wait()

    @pl.loop(0, NCHUNK - 1, step=2)
    def _chunks(c):
        d1 = pltpu.async_copy(g_src(c + 1), rows1, sem1)
        pltpu.async_copy(d_src(c + 2), dst_idx.at[c + 2], sem2)
        pltpu.make_async_copy(g_src(c), rows0, sem0).wait()
        pltpu.sync_copy(rows0, acc.at[dst_idx.at[c]], add=True)
        pltpu.async_copy(g_src(c + 2), rows0, sem0)
        pltpu.async_copy(ones, deg.at[dst_idx.at[c]], sem3, add=True)

        @pl.when(c + 3 < NCHUNK)
        def _():
            pltpu.async_copy(d_src(c + 3), dst_idx.at[c + 3], sem2)

        d1.wait()
        pltpu.sync_copy(rows1, acc.at[dst_idx.at[c + 1]], add=True)
        pltpu.async_copy(ones, deg.at[dst_idx.at[c + 1]], sem3, add=True)

        @pl.when(c > 0)
        def _():
            w_deg(c - 2)
            w_deg(c - 1)

        pltpu.make_async_copy(d_src(c + 2), dst_idx.at[c + 2], sem2).wait()

        @pl.when(c + 3 < NCHUNK)
        def _():
            pltpu.make_async_copy(d_src(c + 3), dst_idx.at[c + 3], sem2).wait()

    pltpu.make_async_copy(g_src(NCHUNK - 1), rows0, sem0).wait()
    pltpu.sync_copy(rows0, acc.at[dst_idx.at[NCHUNK - 1]], add=True)
    pltpu.sync_copy(ones, deg.at[dst_idx.at[NCHUNK - 1]], add=True)
    w_deg(NCHUNK - 3)
    w_deg(NCHUNK - 2)

    plsc.subcore_barrier()
    pltpu.sync_copy(acc.at[pl.ds(base, RPT)],
                    acc_out.at[pl.ds(cid * NPAD + base, RPT)])
    pltpu.sync_copy(deg.at[pl.ds(base, RPT)], deg_out.at[cid, pl.ds(base, RPT)])


def _make_sc_segsum(layer):
    return pl.kernel(
        functools.partial(_sc_segsum_kernel, layer),
        out_type=(
            jax.ShapeDtypeStruct((NC * NPAD, D), jnp.float32),
            jax.ShapeDtypeStruct((NC, NPAD), jnp.float32),
        ),
        mesh=plsc.VectorSubcoreMesh(
            core_axis_name="c", subcore_axis_name="s",
            num_cores=NC, num_subcores=NS,
        ),
        scratch_types=(
            pltpu.VMEM((EPW,), jnp.int32),
            pltpu.VMEM((NCHUNK, CH), jnp.int32),
            pltpu.VMEM((CH, D), jnp.float32),
            pltpu.VMEM((CH, D), jnp.float32),
            pltpu.VMEM((CH,), jnp.float32),
            pltpu.VMEM((RPT,), jnp.float32),
            pltpu.VMEM_SHARED((NPAD, D), jnp.float32),
            pltpu.VMEM_SHARED((NPAD,), jnp.float32),
            pltpu.SemaphoreType.DMA,
            pltpu.SemaphoreType.DMA,
            pltpu.SemaphoreType.DMA,
            pltpu.SemaphoreType.DMA,
        ),
    )


_sc_segsum_l0 = _make_sc_segsum(0)
_sc_segsum_l1 = _make_sc_segsum(1)


def kernel(x, adj, W1, b1, W2, b2):
    adj_flat = adj.reshape(2 * 2 * E)

    h1 = _transform(x, W1, b1)
    acc1, deg1 = _sc_segsum_l0(h1, adj_flat)
    h2 = _combine_mm(h1, acc1, deg1, W2, b2)
    acc2, deg2 = _sc_segsum_l1(h2, adj_flat)
    return _combine_exp(h2, acc2, deg2)
